# baseline (speedup 1.0000x reference)
import functools

import jax
import jax.numpy as jnp
from jax import lax
from jax.experimental import pallas as pl
from jax.experimental.pallas import tpu as pltpu
from jax.experimental.pallas import tpu_sc as plsc

_B = 16384
_ZD = 16
_M_PAD = 1024
_NW = 32
_B_SC = 1024
_B_TC_BLK = 1024
_PPW = _B_SC // _NW
_NBLK = _PPW // 32
_M_TRUE = 1000
_ECH = 100


def _sc_stage1(zt_hbm, e2_hbm, part_hbm, zt_v, e_v, mins_v, row_v, e_sm):
    ci = lax.axis_index("c")
    si = lax.axis_index("s")
    wid = si * 2 + ci

    pltpu.sync_copy(zt_hbm.at[wid], zt_v)
    pltpu.sync_copy(e2_hbm, e_v)

    def cbody(c, _):
        def kbody(k, _):
            er = e_v[pl.ds((c * _ECH + k) * _ZD, _ZD)]
            for j in range(_ZD):
                e_sm[k * _ZD + j] = er[j]
            return _

        lax.fori_loop(0, _ECH, kbody, None)

        for blk in range(_NBLK):
            off = blk * 32
            za = [zt_v[j, pl.ds(off, 16)] for j in range(_ZD)]
            zb = [zt_v[j, pl.ds(off + 16, 16)] for j in range(_ZD)]
            zn_a = za[0] * za[0]
            zn_b = zb[0] * zb[0]
            for j in range(1, _ZD):
                zn_a = zn_a + za[j] * za[j]
                zn_b = zn_b + zb[j] * zb[j]

            def mbody(mm, _, za=za, zb=zb, zn_a=zn_a, zn_b=zn_b, blk=blk):
                ko = mm * _ZD
                s0 = e_sm[ko]
                acc_a = s0 * za[0]
                acc_b = s0 * zb[0]
                for j in range(1, _ZD):
                    sj = e_sm[ko + j]
                    acc_a = acc_a + sj * za[j]
                    acc_b = acc_b + sj * zb[j]
                d = jnp.minimum(acc_a + zn_a, acc_b + zn_b)
                sl = pl.ds((c * _ECH + mm) * 16, 16)
                if blk == 0:
                    mins_v[sl] = d
                else:
                    mins_v[sl] = jnp.minimum(mins_v[sl], d)
                return _

            lax.fori_loop(0, _ECH, mbody, None)
        return _

    lax.fori_loop(0, _M_TRUE // _ECH, cbody, None)

    def rbody(g, _):
        ss = []
        for l in range(16):
            v = mins_v[pl.ds((g * 16 + l) * 16, 16)]
            m0 = jnp.minimum(jnp.minimum(v[0], v[1]),
                             jnp.minimum(v[2], v[3]))
            m1 = jnp.minimum(jnp.minimum(v[4], v[5]),
                             jnp.minimum(v[6], v[7]))
            m2 = jnp.minimum(jnp.minimum(v[8], v[9]),
                             jnp.minimum(v[10], v[11]))
            m3 = jnp.minimum(jnp.minimum(v[12], v[13]),
                             jnp.minimum(v[14], v[15]))
            ss.append(jnp.minimum(jnp.minimum(m0, m1),
                                  jnp.minimum(m2, m3)))
        lane = lax.iota(jnp.int32, 16)
        rv = jnp.full((16,), ss[0], jnp.float32)
        for l in range(1, 16):
            rv = jnp.where(lane == l, jnp.full((16,), ss[l], jnp.float32),
                           rv)
        row_v[pl.ds(g * 16, 16)] = rv
        return _

    lax.fori_loop(0, _M_PAD // 16, rbody, None)
    pltpu.sync_copy(row_v, part_hbm.at[wid])


def _tc_dense(z_ref, et_ref, out_ref, acc_ref):
    i = pl.program_id(0)
    nblk = pl.num_programs(0)

    @pl.when(i == 0)
    def _init():
        acc_ref[...] = jnp.full(acc_ref.shape, jnp.inf, dtype=jnp.float32)

    zv = z_ref[...]
    znorm = jnp.sum(zv * zv, axis=1, keepdims=True)
    zaug = jnp.concatenate([zv, znorm], axis=1)
    d = jax.lax.dot_general(
        zaug, et_ref[...], (((1,), (0,)), ((), ())),
        preferred_element_type=jnp.float32,
        precision=jax.lax.Precision.HIGHEST,
    )
    d8 = jnp.min(d.reshape(_B_TC_BLK // 8, 8, _M_PAD), axis=0)
    acc_ref[...] = jnp.minimum(acc_ref[...], d8)

    @pl.when(i == nblk - 1)
    def _fin():
        out_ref[...] = jnp.min(acc_ref[...], axis=0, keepdims=True)


def _sc_stage2(part_hbm, tc_hbm, et_hbm, out_hbm, part_v, tc_v, et_v, out_v,
               *, m_true):
    ci = lax.axis_index("c")
    si = lax.axis_index("s")

    @pl.when(jnp.logical_and(ci == 0, si == 0))
    def _():
        pltpu.sync_copy(part_hbm, part_v)
        pltpu.sync_copy(tc_hbm, tc_v)
        pltpu.sync_copy(et_hbm, et_v)

        def gbody(g, sacc):
            sl = pl.ds(g * 16, 16)
            mv = jnp.minimum(part_v[0, sl], tc_v[sl])
            for r in range(1, _NW):
                mv = jnp.minimum(mv, part_v[r, sl])
            en = et_v[0, sl] * et_v[0, sl]
            for j in range(1, _ZD):
                en = en + et_v[j, sl] * et_v[j, sl]
            ok = g * 16 + lax.iota(jnp.int32, 16) < m_true
            return sacc + jnp.where(ok, mv + en, 0.0)

        sacc = lax.fori_loop(0, _M_PAD // 16, gbody,
                             jnp.zeros((16,), jnp.float32))
        res = sacc[0]
        for k in range(1, 16):
            res = res + sacc[k]
        res = res * (1.0 / float(m_true))
        out_v[:] = jnp.full((16,), res, jnp.float32)
        pltpu.sync_copy(out_v, out_hbm)


def kernel(z, e, M):
    del M
    m_true = e.shape[0]
    zt = z[:_B_SC].T.reshape(_ZD, _NW, _PPW).swapaxes(0, 1)
    e2 = (-2.0 * e).reshape(-1)
    et = jnp.pad(e, ((0, _M_PAD - m_true), (0, 0))).T
    et1 = jnp.concatenate([-2.0 * et, jnp.ones((1, _M_PAD), jnp.float32)],
                          axis=0)

    mesh = plsc.VectorSubcoreMesh(core_axis_name="c", subcore_axis_name="s")

    k1 = functools.partial(
        pl.kernel,
        mesh=mesh,
        out_type=jax.ShapeDtypeStruct((_NW, _M_PAD), jnp.float32),
        scratch_types=[
            pltpu.VMEM((_ZD, _PPW), jnp.float32),
            pltpu.VMEM((e.shape[0] * _ZD,), jnp.float32),
            pltpu.VMEM((_M_PAD * 16,), jnp.float32),
            pltpu.VMEM((_M_PAD,), jnp.float32),
            pltpu.SMEM((_ECH * _ZD,), jnp.float32),
        ],
    )(_sc_stage1)
    partial = k1(zt, e2)

    tc_row = pl.pallas_call(
        _tc_dense,
        grid=((_B - _B_SC) // _B_TC_BLK,),
        in_specs=[
            pl.BlockSpec((_B_TC_BLK, _ZD), lambda i: (i + _B_SC // _B_TC_BLK, 0)),
            pl.BlockSpec((_ZD + 1, _M_PAD), lambda i: (0, 0)),
        ],
        out_specs=pl.BlockSpec((1, _M_PAD), lambda i: (0, 0)),
        out_shape=jax.ShapeDtypeStruct((1, _M_PAD), jnp.float32),
        scratch_shapes=[pltpu.VMEM((8, _M_PAD), jnp.float32)],
    )(z, et1).reshape(-1)

    k2 = functools.partial(
        pl.kernel,
        mesh=mesh,
        out_type=jax.ShapeDtypeStruct((16,), jnp.float32),
        scratch_types=[
            pltpu.VMEM((_NW, _M_PAD), jnp.float32),
            pltpu.VMEM((_M_PAD,), jnp.float32),
            pltpu.VMEM((_ZD, _M_PAD), jnp.float32),
            pltpu.VMEM((16,), jnp.float32),
        ],
    )(functools.partial(_sc_stage2, m_true=m_true))
    out16 = k2(partial, tc_row, et)
    return out16[0]


# speedup vs baseline: 1.1695x; 1.1695x over previous
import functools

import jax
import jax.numpy as jnp
from jax import lax
from jax.experimental import pallas as pl
from jax.experimental.pallas import tpu as pltpu
from jax.experimental.pallas import tpu_sc as plsc

_B = 16384
_ZD = 16
_M_PAD = 1024
_NW = 32
_B_SC = 1024
_B_TC_BLK = 1024
_PPW = _B_SC // _NW
_NBLK = _PPW // 32
_M_TRUE = 1000
_ECH = 100


def _sc_stage1(zt_hbm, e2_hbm, part_hbm, zt_v, e_v, mins_v, row_v, e_sm):
    ci = lax.axis_index("c")
    si = lax.axis_index("s")
    wid = si * 2 + ci

    pltpu.sync_copy(zt_hbm.at[wid], zt_v)
    pltpu.sync_copy(e2_hbm, e_v)

    def cbody(c, _):
        def kbody(k, _):
            er = e_v[pl.ds((c * _ECH + k) * _ZD, _ZD)]
            for j in range(_ZD):
                e_sm[k * _ZD + j] = er[j]
            return _

        lax.fori_loop(0, _ECH, kbody, None)

        for blk in range(_NBLK):
            off = blk * 32
            za = [zt_v[j, pl.ds(off, 16)] for j in range(_ZD)]
            zb = [zt_v[j, pl.ds(off + 16, 16)] for j in range(_ZD)]
            zn_a = za[0] * za[0]
            zn_b = zb[0] * zb[0]
            for j in range(1, _ZD):
                zn_a = zn_a + za[j] * za[j]
                zn_b = zn_b + zb[j] * zb[j]

            def mbody(mm, _, za=za, zb=zb, zn_a=zn_a, zn_b=zn_b, blk=blk):
                ko = mm * _ZD
                s0 = e_sm[ko]
                acc_a = s0 * za[0]
                acc_b = s0 * zb[0]
                for j in range(1, _ZD):
                    sj = e_sm[ko + j]
                    acc_a = acc_a + sj * za[j]
                    acc_b = acc_b + sj * zb[j]
                d = jnp.minimum(acc_a + zn_a, acc_b + zn_b)
                sl = pl.ds((c * _ECH + mm) * 16, 16)
                if blk == 0:
                    mins_v[sl] = d
                else:
                    mins_v[sl] = jnp.minimum(mins_v[sl], d)
                return _

            lax.fori_loop(0, _ECH, mbody, None)
        return _

    lax.fori_loop(0, _M_TRUE // _ECH, cbody, None)

    def rbody(g, _):
        ss = []
        for l in range(16):
            v = mins_v[pl.ds((g * 16 + l) * 16, 16)]
            m0 = jnp.minimum(jnp.minimum(v[0], v[1]),
                             jnp.minimum(v[2], v[3]))
            m1 = jnp.minimum(jnp.minimum(v[4], v[5]),
                             jnp.minimum(v[6], v[7]))
            m2 = jnp.minimum(jnp.minimum(v[8], v[9]),
                             jnp.minimum(v[10], v[11]))
            m3 = jnp.minimum(jnp.minimum(v[12], v[13]),
                             jnp.minimum(v[14], v[15]))
            ss.append(jnp.minimum(jnp.minimum(m0, m1),
                                  jnp.minimum(m2, m3)))
        lane = lax.iota(jnp.int32, 16)
        rv = jnp.full((16,), ss[0], jnp.float32)
        for l in range(1, 16):
            rv = jnp.where(lane == l, jnp.full((16,), ss[l], jnp.float32),
                           rv)
        row_v[pl.ds(g * 16, 16)] = rv
        return _

    lax.fori_loop(0, _M_PAD // 16, rbody, None)
    pltpu.sync_copy(row_v, part_hbm.at[wid])


def _tc_dense(z_ref, et_ref, out_ref, acc_ref):
    i = pl.program_id(0)
    nblk = pl.num_programs(0)

    @pl.when(i == 0)
    def _init():
        acc_ref[...] = jnp.full(acc_ref.shape, jnp.inf, dtype=jnp.float32)

    zv = z_ref[...]
    znorm = jnp.sum(zv * zv, axis=1, keepdims=True)
    zaug = jnp.concatenate([zv, znorm], axis=1)
    d = jax.lax.dot_general(
        zaug, et_ref[...], (((1,), (0,)), ((), ())),
        preferred_element_type=jnp.float32,
    )
    d8 = jnp.min(d.reshape(_B_TC_BLK // 8, 8, _M_PAD), axis=0)
    acc_ref[...] = jnp.minimum(acc_ref[...], d8)

    @pl.when(i == nblk - 1)
    def _fin():
        out_ref[...] = jnp.min(acc_ref[...], axis=0, keepdims=True)


def _sc_stage2(part_hbm, tc_hbm, et_hbm, out_hbm, part_v, tc_v, et_v, out_v,
               *, m_true):
    ci = lax.axis_index("c")
    si = lax.axis_index("s")

    @pl.when(jnp.logical_and(ci == 0, si == 0))
    def _():
        pltpu.sync_copy(part_hbm, part_v)
        pltpu.sync_copy(tc_hbm, tc_v)
        pltpu.sync_copy(et_hbm, et_v)

        def gbody(g, sacc):
            sl = pl.ds(g * 16, 16)
            mv = jnp.minimum(part_v[0, sl], tc_v[sl])
            for r in range(1, _NW):
                mv = jnp.minimum(mv, part_v[r, sl])
            en = et_v[0, sl] * et_v[0, sl]
            for j in range(1, _ZD):
                en = en + et_v[j, sl] * et_v[j, sl]
            ok = g * 16 + lax.iota(jnp.int32, 16) < m_true
            return sacc + jnp.where(ok, mv + en, 0.0)

        sacc = lax.fori_loop(0, _M_PAD // 16, gbody,
                             jnp.zeros((16,), jnp.float32))
        res = sacc[0]
        for k in range(1, 16):
            res = res + sacc[k]
        res = res * (1.0 / float(m_true))
        out_v[:] = jnp.full((16,), res, jnp.float32)
        pltpu.sync_copy(out_v, out_hbm)


def kernel(z, e, M):
    del M
    m_true = e.shape[0]
    zt = z[:_B_SC].T.reshape(_ZD, _NW, _PPW).swapaxes(0, 1)
    e2 = (-2.0 * e).reshape(-1)
    et = jnp.pad(e, ((0, _M_PAD - m_true), (0, 0))).T
    et1 = jnp.concatenate([-2.0 * et, jnp.ones((1, _M_PAD), jnp.float32)],
                          axis=0)

    mesh = plsc.VectorSubcoreMesh(core_axis_name="c", subcore_axis_name="s")

    k1 = functools.partial(
        pl.kernel,
        mesh=mesh,
        out_type=jax.ShapeDtypeStruct((_NW, _M_PAD), jnp.float32),
        scratch_types=[
            pltpu.VMEM((_ZD, _PPW), jnp.float32),
            pltpu.VMEM((e.shape[0] * _ZD,), jnp.float32),
            pltpu.VMEM((_M_PAD * 16,), jnp.float32),
            pltpu.VMEM((_M_PAD,), jnp.float32),
            pltpu.SMEM((_ECH * _ZD,), jnp.float32),
        ],
    )(_sc_stage1)
    partial = k1(zt, e2)

    tc_row = pl.pallas_call(
        _tc_dense,
        grid=((_B - _B_SC) // _B_TC_BLK,),
        in_specs=[
            pl.BlockSpec((_B_TC_BLK, _ZD), lambda i: (i + _B_SC // _B_TC_BLK, 0)),
            pl.BlockSpec((_ZD + 1, _M_PAD), lambda i: (0, 0)),
        ],
        out_specs=pl.BlockSpec((1, _M_PAD), lambda i: (0, 0)),
        out_shape=jax.ShapeDtypeStruct((1, _M_PAD), jnp.float32),
        scratch_shapes=[pltpu.VMEM((8, _M_PAD), jnp.float32)],
    )(z, et1).reshape(-1)

    k2 = functools.partial(
        pl.kernel,
        mesh=mesh,
        out_type=jax.ShapeDtypeStruct((16,), jnp.float32),
        scratch_types=[
            pltpu.VMEM((_NW, _M_PAD), jnp.float32),
            pltpu.VMEM((_M_PAD,), jnp.float32),
            pltpu.VMEM((_ZD, _M_PAD), jnp.float32),
            pltpu.VMEM((16,), jnp.float32),
        ],
    )(functools.partial(_sc_stage2, m_true=m_true))
    out16 = k2(partial, tc_row, et)
    return out16[0]


# speedup vs baseline: 1.1726x; 1.0026x over previous
import functools

import jax
import jax.numpy as jnp
from jax import lax
from jax.experimental import pallas as pl
from jax.experimental.pallas import tpu as pltpu
from jax.experimental.pallas import tpu_sc as plsc

_B = 16384
_ZD = 16
_M_PAD = 1024
_NW = 32
_B_SC = 1024
_B_TC_BLK = 1024
_PPW = _B_SC // _NW
_NBLK = _PPW // 32
_M_TRUE = 1000
_ECH = 100


def _sc_stage1(zt_hbm, e2_hbm, part_hbm, zt_v, e_v, mins_v, row_v, e_sm):
    ci = lax.axis_index("c")
    si = lax.axis_index("s")
    wid = si * 2 + ci

    pltpu.sync_copy(zt_hbm.at[wid], zt_v)
    pltpu.sync_copy(e2_hbm, e_v)

    def cbody(c, _):
        def kbody(k, _):
            er = e_v[pl.ds((c * _ECH + k) * _ZD, _ZD)]
            for j in range(_ZD):
                e_sm[k * _ZD + j] = er[j]
            return _

        lax.fori_loop(0, _ECH, kbody, None)

        for blk in range(_NBLK):
            off = blk * 32
            za = [zt_v[j, pl.ds(off, 16)] for j in range(_ZD)]
            zb = [zt_v[j, pl.ds(off + 16, 16)] for j in range(_ZD)]
            zn_a = za[0] * za[0]
            zn_b = zb[0] * zb[0]
            for j in range(1, _ZD):
                zn_a = zn_a + za[j] * za[j]
                zn_b = zn_b + zb[j] * zb[j]

            def mbody(mm, _, za=za, zb=zb, zn_a=zn_a, zn_b=zn_b, blk=blk):
                ko = mm * _ZD
                s0 = e_sm[ko]
                acc_a = s0 * za[0]
                acc_b = s0 * zb[0]
                for j in range(1, _ZD):
                    sj = e_sm[ko + j]
                    acc_a = acc_a + sj * za[j]
                    acc_b = acc_b + sj * zb[j]
                d = jnp.minimum(acc_a + zn_a, acc_b + zn_b)
                sl = pl.ds((c * _ECH + mm) * 16, 16)
                if blk == 0:
                    mins_v[sl] = d
                else:
                    mins_v[sl] = jnp.minimum(mins_v[sl], d)
                return _

            lax.fori_loop(0, _ECH, mbody, None)
        return _

    lax.fori_loop(0, _M_TRUE // _ECH, cbody, None)

    def rbody(g, _):
        ss = []
        for l in range(16):
            v = mins_v[pl.ds((g * 16 + l) * 16, 16)]
            m0 = jnp.minimum(jnp.minimum(v[0], v[1]),
                             jnp.minimum(v[2], v[3]))
            m1 = jnp.minimum(jnp.minimum(v[4], v[5]),
                             jnp.minimum(v[6], v[7]))
            m2 = jnp.minimum(jnp.minimum(v[8], v[9]),
                             jnp.minimum(v[10], v[11]))
            m3 = jnp.minimum(jnp.minimum(v[12], v[13]),
                             jnp.minimum(v[14], v[15]))
            ss.append(jnp.minimum(jnp.minimum(m0, m1),
                                  jnp.minimum(m2, m3)))
        lane = lax.iota(jnp.int32, 16)
        rv = jnp.full((16,), ss[0], jnp.float32)
        for l in range(1, 16):
            rv = jnp.where(lane == l, jnp.full((16,), ss[l], jnp.float32),
                           rv)
        row_v[pl.ds(g * 16, 16)] = rv
        return _

    lax.fori_loop(0, _M_PAD // 16, rbody, None)
    pltpu.sync_copy(row_v, part_hbm.at[wid])


def _tc_dense(z_ref, et_ref, out_ref, acc_ref):
    i = pl.program_id(0)
    nblk = pl.num_programs(0)

    @pl.when(i == 0)
    def _init():
        acc_ref[...] = jnp.full(acc_ref.shape, jnp.inf, dtype=jnp.float32)

    zv = z_ref[...]
    g = jax.lax.dot_general(
        zv, et_ref[...], (((1,), (0,)), ((), ())),
        preferred_element_type=jnp.float32,
    )
    znorm = jnp.sum(zv * zv, axis=1, keepdims=True)
    d = znorm + g
    d8 = jnp.min(d.reshape(_B_TC_BLK // 8, 8, _M_PAD), axis=0)
    acc_ref[...] = jnp.minimum(acc_ref[...], d8)

    @pl.when(i == nblk - 1)
    def _fin():
        out_ref[...] = jnp.min(acc_ref[...], axis=0, keepdims=True)


def _sc_stage2(part_hbm, tc_hbm, et_hbm, out_hbm, part_v, tc_v, et_v, out_v,
               *, m_true):
    ci = lax.axis_index("c")
    si = lax.axis_index("s")

    @pl.when(jnp.logical_and(ci == 0, si == 0))
    def _():
        pltpu.sync_copy(part_hbm, part_v)
        pltpu.sync_copy(tc_hbm, tc_v)
        pltpu.sync_copy(et_hbm, et_v)

        def gbody(g, sacc):
            sl = pl.ds(g * 16, 16)
            mv = jnp.minimum(part_v[0, sl], tc_v[sl])
            for r in range(1, _NW):
                mv = jnp.minimum(mv, part_v[r, sl])
            en = et_v[0, sl] * et_v[0, sl]
            for j in range(1, _ZD):
                en = en + et_v[j, sl] * et_v[j, sl]
            ok = g * 16 + lax.iota(jnp.int32, 16) < m_true
            return sacc + jnp.where(ok, mv + en, 0.0)

        sacc = lax.fori_loop(0, _M_PAD // 16, gbody,
                             jnp.zeros((16,), jnp.float32))
        res = sacc[0]
        for k in range(1, 16):
            res = res + sacc[k]
        res = res * (1.0 / float(m_true))
        out_v[:] = jnp.full((16,), res, jnp.float32)
        pltpu.sync_copy(out_v, out_hbm)


def kernel(z, e, M):
    del M
    m_true = e.shape[0]
    zt = z[:_B_SC].T.reshape(_ZD, _NW, _PPW).swapaxes(0, 1)
    e2 = (-2.0 * e).reshape(-1)
    et = jnp.pad(e, ((0, _M_PAD - m_true), (0, 0))).T
    etm2 = -2.0 * et

    mesh = plsc.VectorSubcoreMesh(core_axis_name="c", subcore_axis_name="s")

    k1 = functools.partial(
        pl.kernel,
        mesh=mesh,
        out_type=jax.ShapeDtypeStruct((_NW, _M_PAD), jnp.float32),
        scratch_types=[
            pltpu.VMEM((_ZD, _PPW), jnp.float32),
            pltpu.VMEM((e.shape[0] * _ZD,), jnp.float32),
            pltpu.VMEM((_M_PAD * 16,), jnp.float32),
            pltpu.VMEM((_M_PAD,), jnp.float32),
            pltpu.SMEM((_ECH * _ZD,), jnp.float32),
        ],
    )(_sc_stage1)
    partial = k1(zt, e2)

    tc_row = pl.pallas_call(
        _tc_dense,
        grid=((_B - _B_SC) // _B_TC_BLK,),
        in_specs=[
            pl.BlockSpec((_B_TC_BLK, _ZD), lambda i: (i + _B_SC // _B_TC_BLK, 0)),
            pl.BlockSpec((_ZD, _M_PAD), lambda i: (0, 0)),
        ],
        out_specs=pl.BlockSpec((1, _M_PAD), lambda i: (0, 0)),
        out_shape=jax.ShapeDtypeStruct((1, _M_PAD), jnp.float32),
        scratch_shapes=[pltpu.VMEM((8, _M_PAD), jnp.float32)],
    )(z, etm2).reshape(-1)

    k2 = functools.partial(
        pl.kernel,
        mesh=mesh,
        out_type=jax.ShapeDtypeStruct((16,), jnp.float32),
        scratch_types=[
            pltpu.VMEM((_NW, _M_PAD), jnp.float32),
            pltpu.VMEM((_M_PAD,), jnp.float32),
            pltpu.VMEM((_ZD, _M_PAD), jnp.float32),
            pltpu.VMEM((16,), jnp.float32),
        ],
    )(functools.partial(_sc_stage2, m_true=m_true))
    out16 = k2(partial, tc_row, et)
    return out16[0]
